# baseline (speedup 1.0000x reference)
import jax
import jax.numpy as jnp
from jax.experimental import pallas as pl

_TM = 400


def _proj_body(x_ref, w_ref, o_ref):
    o_ref[...] = jnp.dot(x_ref[...], w_ref[...],
                         preferred_element_type=jnp.float32)


def _layer1_body(adj_ref, s1_ref, b1_ref, w2_ref, s2_ref):
    acc = jnp.dot(adj_ref[...], s1_ref[...],
                  preferred_element_type=jnp.float32)
    h = jnp.maximum(acc + b1_ref[...], 0.0)
    s2_ref[...] = jnp.dot(h, w2_ref[...],
                          preferred_element_type=jnp.float32)


def _layer2_body(adj_ref, s2_ref, b2_ref, o_ref):
    o = jnp.dot(adj_ref[...], s2_ref[...],
                preferred_element_type=jnp.float32) + b2_ref[...]
    m = jnp.max(o, axis=1, keepdims=True)
    lse = m + jnp.log(jnp.sum(jnp.exp(o - m), axis=1, keepdims=True))
    o_ref[...] = o - lse


def kernel(x, adj, W1, b1, W2, b2):
    n, f_in = x.shape
    h_dim = W1.shape[1]
    c_dim = W2.shape[1]
    grid = (n // _TM,)

    s1 = pl.pallas_call(
        _proj_body,
        out_shape=jax.ShapeDtypeStruct((n, h_dim), jnp.float32),
    )(x, W1)

    b1r = b1.reshape(1, h_dim)
    b2r = b2.reshape(1, c_dim)

    s2 = pl.pallas_call(
        _layer1_body,
        grid=grid,
        in_specs=[
            pl.BlockSpec((_TM, n), lambda i: (i, 0)),
            pl.BlockSpec((n, h_dim), lambda i: (0, 0)),
            pl.BlockSpec((1, h_dim), lambda i: (0, 0)),
            pl.BlockSpec((h_dim, c_dim), lambda i: (0, 0)),
        ],
        out_specs=pl.BlockSpec((_TM, c_dim), lambda i: (i, 0)),
        out_shape=jax.ShapeDtypeStruct((n, c_dim), jnp.float32),
    )(adj, s1, b1r, W2)

    out = pl.pallas_call(
        _layer2_body,
        grid=grid,
        in_specs=[
            pl.BlockSpec((_TM, n), lambda i: (i, 0)),
            pl.BlockSpec((n, c_dim), lambda i: (0, 0)),
            pl.BlockSpec((1, c_dim), lambda i: (0, 0)),
        ],
        out_specs=pl.BlockSpec((_TM, c_dim), lambda i: (i, 0)),
        out_shape=jax.ShapeDtypeStruct((n, c_dim), jnp.float32),
    )(adj, s2, b2r)
    return out
